# Initial kernel scaffold; baseline (speedup 1.0000x reference)
import jax
import jax.numpy as jnp
from jax.experimental import pallas as pl


def kernel(embeddings, labels, dataset_ids, prototypes, initialized):
    raise NotImplementedError("write your pallas kernel here")


# speedup vs baseline: 4.4261x; 4.4261x over previous
import functools

import jax
import jax.numpy as jnp
from jax.experimental import pallas as pl
from jax.experimental.pallas import tpu as pltpu

N_DATASETS = 1000
N_CLASSES = 2
DIM = 256
MOMENTUM = 0.99
B = 16384

NSEG = N_DATASETS * N_CLASSES
SEG_PAD = 2048
ROWS = 2048
NB = B // ROWS


def _seg_kernel(emb_ref, seg_ref, proto_ref, initf_ref,
                out_g_ref, out_p_ref, sums_ref, counts_ref):
    i = pl.program_id(0)

    @pl.when(i == 0)
    def _init():
        sums_ref[...] = jnp.zeros_like(sums_ref)
        counts_ref[...] = jnp.zeros_like(counts_ref)

    emb = emb_ref[...]
    seg = seg_ref[0]

    norm = jnp.sqrt(jnp.sum(emb * emb, axis=1, keepdims=True))
    emb_n = emb / jnp.maximum(norm, 1e-12)

    seg_ids = jax.lax.broadcasted_iota(jnp.int32, (SEG_PAD, ROWS), 0)
    onehot = (seg_ids == seg).astype(jnp.float32)
    sums_ref[...] += jnp.dot(onehot, emb_n,
                             preferred_element_type=jnp.float32)
    counts_ref[...] += jnp.sum(onehot, axis=1, keepdims=True)

    @pl.when(i == NB - 1)
    def _epilogue():
        sums = sums_ref[...]
        counts = counts_ref[...]
        protos = proto_ref[...]
        initf = initf_ref[...]

        has = (counts >= 1.0).astype(jnp.float32)
        bp = sums / jnp.maximum(counts, 1.0)
        ema = MOMENTUM * protos + (1.0 - MOMENTUM) * bp
        upd = jnp.where(initf > 0.0, ema, bp)
        newp = jnp.where(has > 0.0, upd, protos)
        out_p_ref[...] = newp[:NSEG, :]

        new_initf = jnp.maximum(initf, has)

        pn_norm = jnp.sqrt(jnp.sum(newp * newp, axis=1, keepdims=True))
        pn = newp / jnp.maximum(pn_norm, 1e-12)

        rows = jax.lax.broadcasted_iota(jnp.int32, (SEG_PAD, 1), 0)
        valid = (rows < NSEG).astype(jnp.float32)
        even = (rows % 2 == 0).astype(jnp.float32) * valid
        odd = (rows % 2 == 1).astype(jnp.float32) * valid

        w0 = new_initf * even
        w1 = new_initf * odd
        num0 = jnp.sum(pn * w0, axis=0, keepdims=True)
        num1 = jnp.sum(pn * w1, axis=0, keepdims=True)
        den0 = jnp.maximum(jnp.sum(w0), 1.0)
        den1 = jnp.maximum(jnp.sum(w1), 1.0)
        g0 = num0 / den0
        g1 = num1 / den1
        g = jnp.concatenate([g0, g1], axis=0)
        g_norm = jnp.sqrt(jnp.sum(g * g, axis=1, keepdims=True))
        out_g_ref[...] = g / jnp.maximum(g_norm, 1e-12)


@jax.jit
def _run(embeddings, seg3, protos2, initf2):
    return pl.pallas_call(
        _seg_kernel,
        grid=(NB,),
        in_specs=[
            pl.BlockSpec((ROWS, DIM), lambda i: (i, 0)),
            pl.BlockSpec((1, 1, ROWS), lambda i: (i, 0, 0)),
            pl.BlockSpec((SEG_PAD, DIM), lambda i: (0, 0)),
            pl.BlockSpec((SEG_PAD, 1), lambda i: (0, 0)),
        ],
        out_specs=[
            pl.BlockSpec((N_CLASSES, DIM), lambda i: (0, 0)),
            pl.BlockSpec((NSEG, DIM), lambda i: (0, 0)),
        ],
        out_shape=[
            jax.ShapeDtypeStruct((N_CLASSES, DIM), jnp.float32),
            jax.ShapeDtypeStruct((NSEG, DIM), jnp.float32),
        ],
        scratch_shapes=[
            pltpu.VMEM((SEG_PAD, DIM), jnp.float32),
            pltpu.VMEM((SEG_PAD, 1), jnp.float32),
        ],
        compiler_params=pltpu.CompilerParams(
            dimension_semantics=("arbitrary",),
        ),
    )(embeddings, seg3, protos2, initf2)


def kernel(embeddings, labels, dataset_ids, prototypes, initialized):
    seg = dataset_ids.astype(jnp.int32) * N_CLASSES + labels.astype(jnp.int32)
    seg3 = seg.reshape(NB, 1, ROWS)
    protos2 = prototypes.reshape(NSEG, DIM)
    protos_pad = jnp.pad(protos2, ((0, SEG_PAD - NSEG), (0, 0)))
    initf = jnp.pad(initialized.reshape(NSEG).astype(jnp.float32),
                    (0, SEG_PAD - NSEG)).reshape(SEG_PAD, 1)
    g, newp = _run(embeddings.astype(jnp.float32), seg3, protos_pad, initf)
    return (g, newp.reshape(N_DATASETS, N_CLASSES, DIM))
